# baseline (speedup 1.0000x reference)
import functools

import jax
import jax.numpy as jnp
from jax import lax
from jax.experimental import pallas as pl
from jax.experimental.pallas import tpu as pltpu
from jax.experimental.pallas import tpu_sc as plsc

B, NA, NB, D = 8, 1000, 1000, 256
N = B * NA
CHUNK = 40
NCHUNK = N // CHUNK
NW = 32
LAN = D // 16
EPS = 1e-12


def _tc_argmax_body(thr_ref, iou_ref, idx_ref, mask_ref):
    x = iou_ref[0]
    col = lax.broadcasted_iota(jnp.int32, (NA, NB), 1)
    mx = jnp.max(x, axis=1, keepdims=True)
    cand = jnp.where(x == mx, col, NB)
    jst = jnp.min(cand, axis=1, keepdims=True)
    b = pl.program_id(0)
    mk = (mx >= thr_ref[0]).astype(jnp.float32)
    idx_ref[...] = (jst + b * NB).T.reshape(1, 1, NA)
    mask_ref[...] = mk.T.reshape(1, 1, NA)


def _sc_dot_body(az_hbm, bp_hbm, idx_hbm, out_hbm,
                 idx_v, az_v, gp_v, o_v, sems):
    wid = lax.axis_index("s") * 2 + lax.axis_index("c")
    n_t = 7
    pltpu.sync_copy(idx_hbm, idx_v)

    def copies(t, g):
        u = t % 2
        r0 = pl.multiple_of(g * CHUNK, 8)
        pltpu.async_copy(bp_hbm.at[idx_v.at[g, 0]], gp_v.at[u], sems.at[u, 0])
        pltpu.async_copy(az_hbm.at[pl.ds(r0, CHUNK)], az_v.at[u], sems.at[u, 1])

    def waits(t):
        u = t % 2
        pltpu.make_async_copy(bp_hbm.at[pl.ds(0, CHUNK)], gp_v.at[u], sems.at[u, 0]).wait()
        pltpu.make_async_copy(az_hbm.at[pl.ds(0, CHUNK)], az_v.at[u], sems.at[u, 1]).wait()

    def compute(t):
        u = t % 2

        def body(r, carry):
            db = jnp.zeros((16,), jnp.float32)
            ngp = jnp.zeros((16,), jnp.float32)
            naz = jnp.zeros((16,), jnp.float32)
            for d in range(LAN):
                sl = pl.ds(16 * d, 16)
                az = az_v[u, r, sl]
                gp = gp_v[u, r, sl]
                db = db + gp * az
                ngp = ngp + gp * gp
                naz = naz + az * az
            o_v[r, pl.ds(0, 16)] = db
            o_v[r, pl.ds(16, 16)] = ngp
            o_v[r, pl.ds(32, 16)] = naz
            return carry

        lax.fori_loop(0, CHUNK, body, 0, unroll=4)

    copies(0, wid)
    for t in range(n_t):
        g = wid + NW * t

        @pl.when(g < NCHUNK)
        def _(t=t):
            waits(t)

        if t + 1 < n_t:
            g2 = wid + NW * (t + 1)

            @pl.when(g2 < NCHUNK)
            def _(t=t, g2=g2):
                copies(t + 1, g2)

        @pl.when(g < NCHUNK)
        def _(t=t, g=g):
            compute(t)
            r0 = pl.multiple_of(g * CHUNK, 8)
            pltpu.sync_copy(o_v, out_hbm.at[pl.ds(r0, CHUNK)])


def _tc_onehot_body(idx_ref, m_ref, ap_ref, bz_ref, sa_ref, acc_ref):
    b = pl.program_id(0)
    jl = idx_ref[0] - b * NB
    m = m_ref[0]
    ap = ap_ref[0]
    bz = bz_ref[0]

    def nrm_rows(x):
        n = jnp.sqrt(jnp.sum(x * x, axis=1, keepdims=True))
        return x / jnp.maximum(n, EPS)

    nap = nrm_rows(ap)
    nbz = nrm_rows(bz)
    rowj = lax.broadcasted_iota(jnp.int32, (NB, NA), 0)
    ot = ((rowj == jnp.broadcast_to(jl, (NB, NA)))
          & (jnp.broadcast_to(m, (NB, NA)) >= 0.5)).astype(jnp.bfloat16)
    hi = nap.astype(jnp.bfloat16)
    lo = (nap - hi.astype(jnp.float32)).astype(jnp.bfloat16)
    dn = (((1,), (0,)), ((), ()))
    w = (lax.dot_general(ot, hi, dn, preferred_element_type=jnp.float32)
         + lax.dot_general(ot, lo, dn, preferred_element_type=jnp.float32))
    sa_b = jnp.sum(w * nbz)

    @pl.when(b == 0)
    def _():
        acc_ref[0] = 0.0

    acc_ref[0] = acc_ref[0] + sa_b

    @pl.when(b == B - 1)
    def _():
        sa_ref[0, 0] = acc_ref[0]


def _tc_final_body(sa_ref, o_ref, m_ref, loss_ref, c_ref):
    o = o_ref[...]
    ei = (lax.broadcasted_iota(jnp.int32, (64, 8), 0) // 16 ==
          lax.broadcasted_iota(jnp.int32, (64, 8), 1)).astype(jnp.float32)
    r = lax.dot_general(ei, o, (((0,), (1,)), ((), ())),
                        precision=lax.Precision.HIGHEST,
                        preferred_element_type=jnp.float32)
    db = r[0:1, :]
    ngp = r[1:2, :]
    naz = r[2:3, :]

    def nrm(x):
        return jnp.maximum(jnp.sqrt(x), EPS)

    pb = db / (nrm(ngp) * nrm(naz))
    sb = jnp.float32(0.0)
    cnt = jnp.float32(0.0)
    for b in range(B):
        m_b = m_ref[b]
        sb = sb + jnp.sum(m_b * pb[:, b * NA:(b + 1) * NA])
        cnt_b = jnp.sum(m_b)
        c_ref[b, 0] = cnt_b
        cnt = cnt + cnt_b
    denom = jnp.maximum(cnt, 1.0)
    total = sb + sa_ref[0, 0]
    loss_ref[0, 0] = -total / (2.0 * denom)


def kernel(feat_a_p, feat_a_z, feat_b_p, feat_b_z, iou, iou_threshold):
    thr = jnp.asarray(iou_threshold, jnp.float32).reshape(1)

    flat_idx, mask = pl.pallas_call(
        _tc_argmax_body,
        grid=(B,),
        in_specs=[
            pl.BlockSpec(memory_space=pltpu.SMEM),
            pl.BlockSpec((1, NA, NB), lambda b: (b, 0, 0)),
        ],
        out_specs=[
            pl.BlockSpec((1, 1, NA), lambda b: (b, 0, 0)),
            pl.BlockSpec((1, 1, NA), lambda b: (b, 0, 0)),
        ],
        out_shape=[
            jax.ShapeDtypeStruct((B, 1, NA), jnp.int32),
            jax.ShapeDtypeStruct((B, 1, NA), jnp.float32),
        ],
    )(thr, iou)

    idx3d = flat_idx.reshape(NCHUNK, 1, CHUNK)

    mesh = plsc.VectorSubcoreMesh(core_axis_name="c", subcore_axis_name="s")
    sc_dots = functools.partial(
        pl.kernel,
        out_type=jax.ShapeDtypeStruct((N, 64), jnp.float32),
        mesh=mesh,
        scratch_types=[
            pltpu.VMEM((NCHUNK, 1, CHUNK), jnp.int32),
            pltpu.VMEM((2, CHUNK, D), jnp.float32),
            pltpu.VMEM((2, CHUNK, D), jnp.float32),
            pltpu.VMEM((CHUNK, 64), jnp.float32),
            pltpu.SemaphoreType.DMA((2, 2)),
        ],
    )(_sc_dot_body)
    packed = sc_dots(
        feat_a_z.reshape(N, D),
        feat_b_p.reshape(B * NB, D),
        idx3d,
    )

    sa_o = pl.pallas_call(
        _tc_onehot_body,
        grid=(B,),
        in_specs=[
            pl.BlockSpec((1, 1, NA), lambda b: (b, 0, 0)),
            pl.BlockSpec((1, 1, NA), lambda b: (b, 0, 0)),
            pl.BlockSpec((1, NA, D), lambda b: (b, 0, 0)),
            pl.BlockSpec((1, NB, D), lambda b: (b, 0, 0)),
        ],
        out_specs=pl.BlockSpec(memory_space=pltpu.SMEM),
        out_shape=jax.ShapeDtypeStruct((1, 1), jnp.float32),
        scratch_shapes=[pltpu.SMEM((1,), jnp.float32)],
    )(flat_idx, mask, feat_a_p, feat_b_z)

    loss_o, cnt_o = pl.pallas_call(
        _tc_final_body,
        grid=(1,),
        in_specs=[
            pl.BlockSpec(memory_space=pltpu.SMEM),
            pl.BlockSpec((N, 64), lambda i: (0, 0)),
            pl.BlockSpec((B, 1, NA), lambda i: (0, 0, 0)),
        ],
        out_specs=[
            pl.BlockSpec(memory_space=pltpu.SMEM),
            pl.BlockSpec(memory_space=pltpu.SMEM),
        ],
        out_shape=[
            jax.ShapeDtypeStruct((1, 1), jnp.float32),
            jax.ShapeDtypeStruct((B, 1), jnp.float32),
        ],
    )(sa_o, packed, mask)

    return (loss_o[0, 0], cnt_o[:, 0])


# speedup vs baseline: 1.0330x; 1.0330x over previous
import functools

import jax
import jax.numpy as jnp
from jax import lax
from jax.experimental import pallas as pl
from jax.experimental.pallas import tpu as pltpu
from jax.experimental.pallas import tpu_sc as plsc

B, NA, NB, D = 8, 1000, 1000, 256
N = B * NA
CHUNK = 40
NCHUNK = N // CHUNK
NW = 32
LAN = D // 16
EPS = 1e-12


def _tc_argmax_body(thr_ref, iou_ref, idx_ref, mask_ref):
    x = iou_ref[0]
    col = lax.broadcasted_iota(jnp.int32, (NA, NB), 1)
    mx = jnp.max(x, axis=1, keepdims=True)
    cand = jnp.where(x == mx, col, NB)
    jst = jnp.min(cand, axis=1, keepdims=True)
    b = pl.program_id(0)
    mk = (mx >= thr_ref[0]).astype(jnp.float32)
    idx_ref[...] = (jst + b * NB).T.reshape(1, 1, NA)
    mask_ref[...] = mk.T.reshape(1, 1, NA)


def _sc_dot_body(az_hbm, bp_hbm, idx_hbm, out_hbm,
                 idx_v, az_v, gp_v, o_v, sems):
    wid = lax.axis_index("s") * 2 + lax.axis_index("c")
    n_t = 7

    def copies(t, g):
        u = t % 2
        pltpu.sync_copy(idx_hbm.at[g], idx_v.at[u])
        r0 = pl.multiple_of(g * CHUNK, 8)
        pltpu.async_copy(bp_hbm.at[idx_v.at[u, 0]], gp_v.at[u], sems.at[u, 0])
        pltpu.async_copy(az_hbm.at[pl.ds(r0, CHUNK)], az_v.at[u], sems.at[u, 1])

    def waits(t):
        u = t % 2
        pltpu.make_async_copy(bp_hbm.at[pl.ds(0, CHUNK)], gp_v.at[u], sems.at[u, 0]).wait()
        pltpu.make_async_copy(az_hbm.at[pl.ds(0, CHUNK)], az_v.at[u], sems.at[u, 1]).wait()

    def compute(t):
        u = t % 2

        def body(r, carry):
            db = jnp.zeros((16,), jnp.float32)
            ngp = jnp.zeros((16,), jnp.float32)
            naz = jnp.zeros((16,), jnp.float32)
            for d in range(LAN):
                sl = pl.ds(16 * d, 16)
                az = az_v[u, r, sl]
                gp = gp_v[u, r, sl]
                db = db + gp * az
                ngp = ngp + gp * gp
                naz = naz + az * az
            o_v[r, pl.ds(0, 16)] = db
            o_v[r, pl.ds(16, 16)] = ngp
            o_v[r, pl.ds(32, 16)] = naz
            return carry

        lax.fori_loop(0, CHUNK, body, 0, unroll=4)

    copies(0, wid)
    for t in range(n_t):
        g = wid + NW * t

        @pl.when(g < NCHUNK)
        def _(t=t):
            waits(t)

        if t + 1 < n_t:
            g2 = wid + NW * (t + 1)

            @pl.when(g2 < NCHUNK)
            def _(t=t, g2=g2):
                copies(t + 1, g2)

        @pl.when(g < NCHUNK)
        def _(t=t, g=g):
            compute(t)
            r0 = pl.multiple_of(g * CHUNK, 8)
            pltpu.sync_copy(o_v, out_hbm.at[pl.ds(r0, CHUNK)])


def _tc_onehot_body(idx_ref, m_ref, ap_ref, bz_ref, sa_ref, acc_ref):
    b = pl.program_id(0)
    jl = idx_ref[0] - b * NB
    m = m_ref[0]
    ap = ap_ref[0]
    bz = bz_ref[0]

    def nrm_rows(x):
        n = jnp.sqrt(jnp.sum(x * x, axis=1, keepdims=True))
        return x / jnp.maximum(n, EPS)

    nap = nrm_rows(ap)
    nbz = nrm_rows(bz)
    rowj = lax.broadcasted_iota(jnp.int32, (NB, NA), 0)
    ot = ((rowj == jnp.broadcast_to(jl, (NB, NA)))
          & (jnp.broadcast_to(m, (NB, NA)) >= 0.5)).astype(jnp.bfloat16)
    hi = nap.astype(jnp.bfloat16)
    lo = (nap - hi.astype(jnp.float32)).astype(jnp.bfloat16)
    dn = (((1,), (0,)), ((), ()))
    w = (lax.dot_general(ot, hi, dn, preferred_element_type=jnp.float32)
         + lax.dot_general(ot, lo, dn, preferred_element_type=jnp.float32))
    sa_b = jnp.sum(w * nbz)

    @pl.when(b == 0)
    def _():
        acc_ref[0] = 0.0

    acc_ref[0] = acc_ref[0] + sa_b

    @pl.when(b == B - 1)
    def _():
        sa_ref[0, 0] = acc_ref[0]


def _tc_final_body(sa_ref, o_ref, m_ref, loss_ref, c_ref):
    o = o_ref[...]
    ei = (lax.broadcasted_iota(jnp.int32, (64, 8), 0) // 16 ==
          lax.broadcasted_iota(jnp.int32, (64, 8), 1)).astype(jnp.float32)
    r = lax.dot_general(ei, o, (((0,), (1,)), ((), ())),
                        precision=lax.Precision.HIGHEST,
                        preferred_element_type=jnp.float32)
    db = r[0:1, :]
    ngp = r[1:2, :]
    naz = r[2:3, :]

    def nrm(x):
        return jnp.maximum(jnp.sqrt(x), EPS)

    pb = db / (nrm(ngp) * nrm(naz))
    sb = jnp.float32(0.0)
    cnt = jnp.float32(0.0)
    for b in range(B):
        m_b = m_ref[b]
        sb = sb + jnp.sum(m_b * pb[:, b * NA:(b + 1) * NA])
        cnt_b = jnp.sum(m_b)
        c_ref[b, 0] = cnt_b
        cnt = cnt + cnt_b
    denom = jnp.maximum(cnt, 1.0)
    total = sb + sa_ref[0, 0]
    loss_ref[0, 0] = -total / (2.0 * denom)


def kernel(feat_a_p, feat_a_z, feat_b_p, feat_b_z, iou, iou_threshold):
    thr = jnp.asarray(iou_threshold, jnp.float32).reshape(1)

    flat_idx, mask = pl.pallas_call(
        _tc_argmax_body,
        grid=(B,),
        in_specs=[
            pl.BlockSpec(memory_space=pltpu.SMEM),
            pl.BlockSpec((1, NA, NB), lambda b: (b, 0, 0)),
        ],
        out_specs=[
            pl.BlockSpec((1, 1, NA), lambda b: (b, 0, 0)),
            pl.BlockSpec((1, 1, NA), lambda b: (b, 0, 0)),
        ],
        out_shape=[
            jax.ShapeDtypeStruct((B, 1, NA), jnp.int32),
            jax.ShapeDtypeStruct((B, 1, NA), jnp.float32),
        ],
    )(thr, iou)

    idx3d = flat_idx.reshape(NCHUNK, 1, CHUNK)

    mesh = plsc.VectorSubcoreMesh(core_axis_name="c", subcore_axis_name="s")
    sc_dots = functools.partial(
        pl.kernel,
        out_type=jax.ShapeDtypeStruct((N, 64), jnp.float32),
        mesh=mesh,
        scratch_types=[
            pltpu.VMEM((2, 1, CHUNK), jnp.int32),
            pltpu.VMEM((2, CHUNK, D), jnp.float32),
            pltpu.VMEM((2, CHUNK, D), jnp.float32),
            pltpu.VMEM((CHUNK, 64), jnp.float32),
            pltpu.SemaphoreType.DMA((2, 2)),
        ],
    )(_sc_dot_body)
    packed = sc_dots(
        feat_a_z.reshape(N, D),
        feat_b_p.reshape(B * NB, D),
        idx3d,
    )

    sa_o = pl.pallas_call(
        _tc_onehot_body,
        grid=(B,),
        in_specs=[
            pl.BlockSpec((1, 1, NA), lambda b: (b, 0, 0)),
            pl.BlockSpec((1, 1, NA), lambda b: (b, 0, 0)),
            pl.BlockSpec((1, NA, D), lambda b: (b, 0, 0)),
            pl.BlockSpec((1, NB, D), lambda b: (b, 0, 0)),
        ],
        out_specs=pl.BlockSpec(memory_space=pltpu.SMEM),
        out_shape=jax.ShapeDtypeStruct((1, 1), jnp.float32),
        scratch_shapes=[pltpu.SMEM((1,), jnp.float32)],
    )(flat_idx, mask, feat_a_p, feat_b_z)

    loss_o, cnt_o = pl.pallas_call(
        _tc_final_body,
        grid=(1,),
        in_specs=[
            pl.BlockSpec(memory_space=pltpu.SMEM),
            pl.BlockSpec((N, 64), lambda i: (0, 0)),
            pl.BlockSpec((B, 1, NA), lambda i: (0, 0, 0)),
        ],
        out_specs=[
            pl.BlockSpec(memory_space=pltpu.SMEM),
            pl.BlockSpec(memory_space=pltpu.SMEM),
        ],
        out_shape=[
            jax.ShapeDtypeStruct((1, 1), jnp.float32),
            jax.ShapeDtypeStruct((B, 1), jnp.float32),
        ],
    )(sa_o, packed, mask)

    return (loss_o[0, 0], cnt_o[:, 0])
